# Initial kernel scaffold; baseline (speedup 1.0000x reference)
import functools

import jax
import jax.numpy as jnp
from jax import lax
from jax.experimental import pallas as pl
from jax.experimental.pallas import tpu as pltpu
from jax.experimental.pallas import tpu_sc as plsc

NC = 2
NS = 16
L = 16

CH = 128
DW = 144
PAD_ROWS = 1264


def _matmul_tc(A, W):
    M, K = A.shape
    Q = W.shape[1]
    BM = 400

    def body(a_ref, w_ref, o_ref):
        o_ref[...] = jnp.dot(a_ref[...], w_ref[...],
                             preferred_element_type=jnp.float32)

    return pl.pallas_call(
        body,
        grid=(M // BM,),
        in_specs=[pl.BlockSpec((BM, K), lambda i: (i, 0)),
                  pl.BlockSpec((K, Q), lambda i: (0, 0))],
        out_specs=pl.BlockSpec((BM, Q), lambda i: (i, 0)),
        out_shape=jax.ShapeDtypeStruct((M, Q), jnp.float32),
    )(A, W)


def _segsum_sc(Y2, src, dst, w, n_nodes):
    N = n_nodes
    E_pad = src.shape[0]
    per_tile = E_pad // NS
    n_chunks = per_tile // CH
    NPAD = N + PAD_ROWS
    zrows_per_tile = NPAD // NS
    orows_per_tile = N // NS

    mesh = plsc.VectorSubcoreMesh(core_axis_name="c", subcore_axis_name="s")

    @functools.partial(
        pl.kernel,
        out_type=jax.ShapeDtypeStruct((2 * N, DW), jnp.float32),
        mesh=mesh,
        scratch_types=[
            pltpu.VMEM((CH,), jnp.int32),
            pltpu.VMEM((CH,), jnp.int32),
            pltpu.VMEM((CH,), jnp.float32),
            pltpu.VMEM((CH, 128), jnp.float32),
            pltpu.VMEM((CH, DW), jnp.float32),
            pltpu.VMEM_SHARED((NPAD, DW), jnp.float32),
            pltpu.SemaphoreType.DMA,
        ],
    )
    def k(y_hbm, src_hbm, dst_hbm, w_hbm, out_hbm,
          sidx, didx, wv, rows, srows, acc, sem):
        c = lax.axis_index("c")
        s = lax.axis_index("s")
        zeros_l = jnp.zeros((L,), jnp.float32)
        onehot = (lax.iota(jnp.int32, L) == 0).astype(jnp.float32)

        def zrow(i, carry):
            for v in range(DW // L):
                srows[i, pl.ds(v * L, L)] = zeros_l
            return carry
        lax.fori_loop(0, CH, zrow, 0)
        zbase = s * zrows_per_tile
        nfull = zrows_per_tile // CH
        rem = zrows_per_tile % CH
        for j in range(nfull):
            pltpu.sync_copy(srows, acc.at[pl.ds(zbase + j * CH, CH)])
        if rem:
            pltpu.sync_copy(srows.at[pl.ds(0, rem)],
                            acc.at[pl.ds(zbase + nfull * CH, rem)])
        plsc.subcore_barrier()

        ebase = s * per_tile
        coff = c * N

        def chunk_body(i, carry):
            off = ebase + i * CH
            pltpu.sync_copy(src_hbm.at[pl.ds(off, CH)], sidx)
            pltpu.sync_copy(dst_hbm.at[pl.ds(off, CH)], didx)
            pltpu.sync_copy(w_hbm.at[pl.ds(off, CH)], wv)
            for g in range(CH // L):
                sidx[pl.ds(g * L, L)] = sidx[pl.ds(g * L, L)] + coff
            pltpu.async_copy(y_hbm.at[sidx], rows, sem).wait()

            def scale_group(g, carry2):
                wg = wv[pl.ds(g * L, L)]
                for j in range(L):
                    wj = jnp.take(wg, jnp.full((L,), j, jnp.int32),
                                  mode="promise_in_bounds")
                    e = g * L + j
                    for v in range(128 // L):
                        srows[e, pl.ds(v * L, L)] = rows[e, pl.ds(v * L, L)] * wj
                    srows[e, pl.ds(128, L)] = onehot
                return carry2
            lax.fori_loop(0, CH // L, scale_group, 0)

            pltpu.sync_copy(srows, acc.at[didx], add=True)
            return carry
        lax.fori_loop(0, n_chunks, chunk_body, 0)
        plsc.subcore_barrier()

        obase = s * orows_per_tile
        pltpu.sync_copy(acc.at[pl.ds(obase, orows_per_tile)],
                        out_hbm.at[pl.ds(coff + obase, orows_per_tile)])

    return k(Y2, src, dst, w)


def _finish_tc(AGG, S, n_nodes):
    N = n_nodes
    BM = 400
    nb = N // BM

    def body(alo_ref, ahi_ref, s_ref, o_ref):
        alo = alo_ref[...]
        ahi = ahi_ref[...]
        deg = jnp.maximum(alo[:, 128:129], 1.0)
        mean = jnp.concatenate([alo[:, :128], ahi[:, :128]], axis=1) / deg
        o_ref[...] = jnp.tanh(mean + s_ref[...])

    return pl.pallas_call(
        body,
        grid=(nb,),
        in_specs=[pl.BlockSpec((BM, DW), lambda i: (i, 0)),
                  pl.BlockSpec((BM, DW), lambda i: (i + nb, 0)),
                  pl.BlockSpec((BM, 256), lambda i: (i, 0))],
        out_specs=pl.BlockSpec((BM, 256), lambda i: (i, 0)),
        out_shape=jax.ShapeDtypeStruct((N, 256), jnp.float32),
    )(AGG, AGG, S)


def kernel(X, edge_index, edge_weight, H, Wi_neigh, Wi_self, bi,
           Wh_neigh, Wh_self, bh):
    N, D = X.shape
    E = edge_weight.shape[0]

    cat = jnp.concatenate([X, H], axis=1)
    W2 = jnp.concatenate(
        [jnp.concatenate([Wi_neigh, Wi_self], axis=1),
         jnp.concatenate([Wh_neigh, Wh_self], axis=1)], axis=0)
    b = bi + bh

    Z = _matmul_tc(cat, W2)
    Y2 = jnp.concatenate([Z[:, :128], Z[:, 128:256]], axis=0)
    S = Z[:, 256:] + b[None, :]

    E_pad = ((E + NS * CH - 1) // (NS * CH)) * (NS * CH)
    pad = E_pad - E
    src = edge_index[0]
    dst = edge_index[1]
    if pad:
        ar = jnp.arange(pad, dtype=jnp.int32)
        src = jnp.concatenate([src, ar % N])
        dst = jnp.concatenate([dst, N + (ar % PAD_ROWS)])
        w = jnp.concatenate([edge_weight, jnp.zeros((pad,), jnp.float32)])
    else:
        w = edge_weight

    AGG = _segsum_sc(Y2, src, dst, w, N)
    return _finish_tc(AGG, S, N)


# speedup vs baseline: 4.9719x; 4.9719x over previous
import functools

import jax
import jax.numpy as jnp
from jax import lax
from jax.experimental import pallas as pl
from jax.experimental.pallas import tpu as pltpu
from jax.experimental.pallas import tpu_sc as plsc

NC = 2
NS = 16
L = 16

CH = 128
N2 = 10240


def _matmul_tc(A, W):
    M, K = A.shape
    Q = W.shape[1]
    BM = 512

    def body(a_ref, w_ref, o_ref):
        o_ref[...] = jnp.dot(a_ref[...], w_ref[...],
                             preferred_element_type=jnp.float32)

    return pl.pallas_call(
        body,
        grid=(M // BM,),
        in_specs=[pl.BlockSpec((BM, K), lambda i: (i, 0)),
                  pl.BlockSpec((K, Q), lambda i: (0, 0))],
        out_specs=pl.BlockSpec((BM, Q), lambda i: (i, 0)),
        out_shape=jax.ShapeDtypeStruct((M, Q), jnp.float32),
    )(A, W)


def _segsum_sc(Y2, src, dst, w):
    E_pad = src.shape[0]
    per_tile = E_pad // NS
    n_chunks = per_tile // CH
    zrows_per_tile = N2 // NS

    mesh = plsc.VectorSubcoreMesh(core_axis_name="c", subcore_axis_name="s")

    @functools.partial(
        pl.kernel,
        out_type=(jax.ShapeDtypeStruct((2 * N2, 128), jnp.float32),
                  jax.ShapeDtypeStruct((NC * NS * N2,), jnp.float32)),
        mesh=mesh,
        scratch_types=[
            pltpu.VMEM((CH,), jnp.int32),
            pltpu.VMEM((CH,), jnp.int32),
            pltpu.VMEM((CH,), jnp.float32),
            pltpu.VMEM((CH, 128), jnp.float32),
            pltpu.VMEM((CH, 128), jnp.float32),
            pltpu.VMEM((N2,), jnp.float32),
            pltpu.VMEM_SHARED((N2, 128), jnp.float32),
            pltpu.SemaphoreType.DMA,
        ],
    )
    def k(y_hbm, src_hbm, dst_hbm, w_hbm, agg_hbm, hist_hbm,
          sidx, didx, wv, rows, srows, hist, acc, sem):
        c = lax.axis_index("c")
        s = lax.axis_index("s")

        def zhist(i, carry):
            hist[pl.ds(i * L, L)] = jnp.zeros((L,), jnp.float32)
            return carry
        lax.fori_loop(0, N2 // L, zhist, 0)

        dper = per_tile // NC
        dbase = s * per_tile + c * dper

        def dchunk(i, carry):
            pltpu.sync_copy(dst_hbm.at[pl.ds(dbase + i * CH, CH)], didx)

            def dgroup(g, carry2):
                dv = didx[pl.ds(g * L, L)]
                for j in range(L):
                    d = dv[j]
                    row = (d // L) * L
                    lane = d - row
                    m = (lax.iota(jnp.int32, L)
                         == jnp.full((L,), lane, jnp.int32))
                    oh = jnp.where(m, jnp.ones((L,), jnp.float32),
                                   jnp.zeros((L,), jnp.float32))
                    plsc.addupdate(hist.at[pl.ds(row, L)], oh)
                return carry2
            lax.fori_loop(0, CH // L, dgroup, 0)
            return carry
        lax.fori_loop(0, dper // CH, dchunk, 0)
        pltpu.sync_copy(hist, hist_hbm.at[pl.ds((c * NS + s) * N2, N2)])

        def zrow(i, carry):
            for v in range(128 // L):
                srows[i, pl.ds(v * L, L)] = jnp.zeros((L,), jnp.float32)
            return carry
        lax.fori_loop(0, CH, zrow, 0)
        zbase = s * zrows_per_tile
        for j in range(zrows_per_tile // CH):
            pltpu.sync_copy(srows, acc.at[pl.ds(zbase + j * CH, CH)])
        plsc.subcore_barrier()

        ebase = s * per_tile
        coff = c * N2

        def chunk_body(i, carry):
            off = ebase + i * CH
            pltpu.sync_copy(src_hbm.at[pl.ds(off, CH)], sidx)
            pltpu.sync_copy(dst_hbm.at[pl.ds(off, CH)], didx)
            pltpu.sync_copy(w_hbm.at[pl.ds(off, CH)], wv)
            for g in range(CH // L):
                sidx[pl.ds(g * L, L)] = sidx[pl.ds(g * L, L)] + coff
            pltpu.async_copy(y_hbm.at[sidx], rows, sem).wait()

            def scale_group(g, carry2):
                wg = wv[pl.ds(g * L, L)]
                for j in range(L):
                    e = g * L + j
                    wjv = jnp.full((L,), wg[j], jnp.float32)
                    for v in range(128 // L):
                        srows[e, pl.ds(v * L, L)] = rows[e, pl.ds(v * L, L)] * wjv
                return carry2
            lax.fori_loop(0, CH // L, scale_group, 0)

            pltpu.sync_copy(srows, acc.at[didx], add=True)
            return carry
        lax.fori_loop(0, n_chunks, chunk_body, 0)
        plsc.subcore_barrier()

        obase = s * zrows_per_tile
        pltpu.sync_copy(acc.at[pl.ds(obase, zrows_per_tile)],
                        agg_hbm.at[pl.ds(coff + obase, zrows_per_tile)])

    return k(Y2, src, dst, w)


def _finish_tc(AGG, HIST, S):
    BM = 512
    nb = N2 // BM
    NW = NC * NS

    def body(alo_ref, ahi_ref, h_ref, s_ref, o_ref):
        deg = jnp.sum(h_ref[...], axis=0).reshape(BM, 1)
        deg = jnp.maximum(deg, 1.0)
        mean = jnp.concatenate([alo_ref[...], ahi_ref[...]], axis=1) / deg
        o_ref[...] = jnp.tanh(mean + s_ref[...])

    return pl.pallas_call(
        body,
        grid=(nb,),
        in_specs=[pl.BlockSpec((BM, 128), lambda i: (i, 0)),
                  pl.BlockSpec((BM, 128), lambda i: (i + nb, 0)),
                  pl.BlockSpec((NW, BM), lambda i: (0, i)),
                  pl.BlockSpec((BM, 256), lambda i: (i, 0))],
        out_specs=pl.BlockSpec((BM, 256), lambda i: (i, 0)),
        out_shape=jax.ShapeDtypeStruct((N2, 256), jnp.float32),
    )(AGG, AGG, HIST, S)


def kernel(X, edge_index, edge_weight, H, Wi_neigh, Wi_self, bi,
           Wh_neigh, Wh_self, bh):
    N, D = X.shape
    E = edge_weight.shape[0]

    zpad = jnp.zeros((N2 - N, D), jnp.float32)
    cat = jnp.concatenate(
        [jnp.concatenate([X, zpad], axis=0),
         jnp.concatenate([H, zpad], axis=0)], axis=1)
    W2 = jnp.concatenate(
        [jnp.concatenate([Wi_neigh, Wi_self], axis=1),
         jnp.concatenate([Wh_neigh, Wh_self], axis=1)], axis=0)
    b = bi + bh

    Z = _matmul_tc(cat, W2)
    Y2 = jnp.concatenate([Z[:, :128], Z[:, 128:256]], axis=0)
    S = Z[:, 256:] + b[None, :]

    quantum = NC * NS * CH
    E_pad = ((E + quantum - 1) // quantum) * quantum
    pad = E_pad - E
    src = edge_index[0]
    dst = edge_index[1]
    if pad:
        ar = jnp.arange(pad, dtype=jnp.int32)
        src = jnp.concatenate([src, ar % N])
        dst = jnp.concatenate([dst, N + (ar % (N2 - N))])
        w = jnp.concatenate([edge_weight, jnp.zeros((pad,), jnp.float32)])
    else:
        w = edge_weight

    AGG, HIST = _segsum_sc(Y2, src, dst, w)
    out = _finish_tc(AGG, HIST.reshape(NC * NS, N2), S)
    return out[:N]


# speedup vs baseline: 7.8310x; 1.5751x over previous
import functools

import jax
import jax.numpy as jnp
from jax import lax
from jax.experimental import pallas as pl
from jax.experimental.pallas import tpu as pltpu
from jax.experimental.pallas import tpu_sc as plsc

NC = 2
NS = 16
L = 16

CH = 128
N2 = 10240


def _matmul_tc(A, W):
    M, K = A.shape
    Q = W.shape[1]
    BM = 512

    def body(a_ref, w_ref, o_ref):
        o_ref[...] = jnp.dot(a_ref[...], w_ref[...],
                             preferred_element_type=jnp.float32)

    return pl.pallas_call(
        body,
        grid=(M // BM,),
        in_specs=[pl.BlockSpec((BM, K), lambda i: (i, 0)),
                  pl.BlockSpec((K, Q), lambda i: (0, 0))],
        out_specs=pl.BlockSpec((BM, Q), lambda i: (i, 0)),
        out_shape=jax.ShapeDtypeStruct((M, Q), jnp.float32),
    )(A, W)


def _segsum_sc(Y2, src2, dst2, w2):
    R = src2.shape[0] - 8
    rows_pt = R // NS
    G = rows_pt // 8
    zrows_per_tile = N2 // NS

    mesh = plsc.VectorSubcoreMesh(core_axis_name="c", subcore_axis_name="s")

    @functools.partial(
        pl.kernel,
        out_type=(jax.ShapeDtypeStruct((2 * N2, 128), jnp.float32),
                  jax.ShapeDtypeStruct((NC * NS * (N2 // CH), 128),
                                       jnp.float32)),
        mesh=mesh,
        scratch_types=[
            pltpu.VMEM((8, 128), jnp.int32),
            pltpu.VMEM((8, 128), jnp.int32),
            pltpu.VMEM((8, 128), jnp.int32),
            pltpu.VMEM((8, 128), jnp.int32),
            pltpu.VMEM((8, 128), jnp.float32),
            pltpu.VMEM((8, 128), jnp.float32),
            pltpu.VMEM((CH, 128), jnp.float32),
            pltpu.VMEM((CH, 128), jnp.float32),
            pltpu.VMEM_SHARED((N2, 128), jnp.float32),
            pltpu.SemaphoreType.DMA,
            pltpu.SemaphoreType.DMA,
            pltpu.SemaphoreType.DMA,
            pltpu.SemaphoreType.DMA,
        ],
    )
    def k(y_hbm, src_hbm, dst_hbm, w_hbm, agg_hbm, hist_hbm,
          sA, sB, dA, dB, wA, wB, rowsE, rowsO, acc,
          semStA, semStB, semG, semS):
        c = lax.axis_index("c")
        s = lax.axis_index("s")
        tbase = s * rows_pt

        hrows = N2 // CH

        def zrowE(i, carry):
            for v in range(128 // L):
                rowsE[i, pl.ds(v * L, L)] = jnp.zeros((L,), jnp.float32)
            return carry
        lax.fori_loop(0, CH, zrowE, 0)

        dgbase = tbase + c * (rows_pt // NC)

        def dchunk(i, carry):
            pltpu.async_copy(dst_hbm.at[pl.ds(dgbase + i * 8, 8)], dA,
                             semStA).wait()

            def drow(jj, carry1):
                def dgroup(g, carry2):
                    dv = dA[jj, pl.ds(g * L, L)]
                    for j in range(L):
                        d = dv[j]
                        r0 = d // CH
                        rem = d - r0 * CH
                        cg = (rem // L) * L
                        lane = rem - cg
                        m = (lax.iota(jnp.int32, L)
                             == jnp.full((L,), lane, jnp.int32))
                        oh = jnp.where(m, jnp.ones((L,), jnp.float32),
                                       jnp.zeros((L,), jnp.float32))
                        plsc.addupdate(rowsE.at[r0, pl.ds(cg, L)], oh)
                    return carry2
                lax.fori_loop(0, CH // L, dgroup, 0)
                return carry1
            lax.fori_loop(0, 8, drow, 0)
            return carry
        lax.fori_loop(0, rows_pt // NC // 8, dchunk, 0)
        pltpu.sync_copy(rowsE.at[pl.ds(0, hrows)],
                        hist_hbm.at[pl.ds((c * NS + s) * hrows, hrows)])

        lax.fori_loop(0, hrows, zrowE, 0)
        zbase = s * zrows_per_tile
        for j in range(zrows_per_tile // CH):
            pltpu.sync_copy(rowsE, acc.at[pl.ds(zbase + j * CH, CH)])
        plsc.subcore_barrier()

        coff = c * N2

        def fire_stage(g, bufs, sem):
            si, di, wi = bufs
            base = tbase + g * 8
            pltpu.async_copy(src_hbm.at[pl.ds(base, 8)], si, sem)
            pltpu.async_copy(dst_hbm.at[pl.ds(base, 8)], di, sem)
            pltpu.async_copy(w_hbm.at[pl.ds(base, 8)], wi, sem)

        def drain_stage(bufs, sem):
            si, di, wi = bufs
            pltpu.make_async_copy(src_hbm.at[pl.ds(0, 8)], si, sem).wait()
            pltpu.make_async_copy(dst_hbm.at[pl.ds(0, 8)], di, sem).wait()
            pltpu.make_async_copy(w_hbm.at[pl.ds(0, 8)], wi, sem).wait()

        def scale(j, rows, wi):
            def scale_group(g, carry2):
                wg = wi[j, pl.ds(g * L, L)]
                for jj in range(L):
                    e = g * L + jj
                    wjv = jnp.full((L,), wg[jj], jnp.float32)
                    for v in range(128 // L):
                        rows[e, pl.ds(v * L, L)] = rows[e, pl.ds(v * L, L)] * wjv
                return carry2
            lax.fori_loop(0, CH // L, scale_group, 0)

        def process_group(bufs):
            si, di, wi = bufs
            for jj in range(8):
                for g in range(CH // L):
                    si[jj, pl.ds(g * L, L)] = si[jj, pl.ds(g * L, L)] + coff

            def pair(q, carry):
                je = 2 * q
                jo = 2 * q + 1
                gE = pltpu.async_copy(y_hbm.at[si.at[je]], rowsE, semG)
                gO = pltpu.async_copy(y_hbm.at[si.at[jo]], rowsO, semG)
                gE.wait()
                gO.wait()
                scale(je, rowsE, wi)
                sE = pltpu.async_copy(rowsE, acc.at[di.at[je]], semS, add=True)
                scale(jo, rowsO, wi)
                sO = pltpu.async_copy(rowsO, acc.at[di.at[jo]], semS, add=True)
                sE.wait()
                sO.wait()
                return carry
            lax.fori_loop(0, 4, pair, 0)

        bufsA = (sA, dA, wA)
        bufsB = (sB, dB, wB)
        fire_stage(0, bufsA, semStA)

        def body(p, carry):
            fire_stage(2 * p + 1, bufsB, semStB)
            drain_stage(bufsA, semStA)
            process_group(bufsA)
            fire_stage(2 * p + 2, bufsA, semStA)
            drain_stage(bufsB, semStB)
            process_group(bufsB)
            return carry
        lax.fori_loop(0, G // 2, body, 0)
        drain_stage(bufsA, semStA)
        plsc.subcore_barrier()

        obase = s * zrows_per_tile
        pltpu.sync_copy(acc.at[pl.ds(obase, zrows_per_tile)],
                        agg_hbm.at[pl.ds(coff + obase, zrows_per_tile)])

    return k(Y2, src2, dst2, w2)


def _finish_tc(AGG, HIST, S):
    BM = 512
    nb = N2 // BM
    NW = NC * NS

    def body(alo_ref, ahi_ref, h_ref, s_ref, o_ref):
        deg = jnp.sum(h_ref[...], axis=0).reshape(BM, 1)
        deg = jnp.maximum(deg, 1.0)
        mean = jnp.concatenate([alo_ref[...], ahi_ref[...]], axis=1) / deg
        o_ref[...] = jnp.tanh(mean + s_ref[...])

    return pl.pallas_call(
        body,
        grid=(nb,),
        in_specs=[pl.BlockSpec((BM, 128), lambda i: (i, 0)),
                  pl.BlockSpec((BM, 128), lambda i: (i + nb, 0)),
                  pl.BlockSpec((NW, BM), lambda i: (0, i)),
                  pl.BlockSpec((BM, 256), lambda i: (i, 0))],
        out_specs=pl.BlockSpec((BM, 256), lambda i: (i, 0)),
        out_shape=jax.ShapeDtypeStruct((N2, 256), jnp.float32),
    )(AGG, AGG, HIST, S)


def kernel(X, edge_index, edge_weight, H, Wi_neigh, Wi_self, bi,
           Wh_neigh, Wh_self, bh):
    N, D = X.shape
    E = edge_weight.shape[0]

    zpad = jnp.zeros((N2 - N, D), jnp.float32)
    cat = jnp.concatenate(
        [jnp.concatenate([X, zpad], axis=0),
         jnp.concatenate([H, zpad], axis=0)], axis=1)
    W2 = jnp.concatenate(
        [jnp.concatenate([Wi_neigh, Wi_self], axis=1),
         jnp.concatenate([Wh_neigh, Wh_self], axis=1)], axis=0)
    b = bi + bh

    Z = _matmul_tc(cat, W2)
    Y2 = jnp.concatenate([Z[:, :128], Z[:, 128:256]], axis=0)
    S = Z[:, 256:] + b[None, :]

    quantum = NS * CH * 16
    E_pad = ((E + quantum - 1) // quantum) * quantum
    pad = E_pad + 8 * CH - E
    src = edge_index[0]
    dst = edge_index[1]
    ar = jnp.arange(pad, dtype=jnp.int32)
    src = jnp.concatenate([src, ar % N]).reshape(-1, CH)
    dst = jnp.concatenate([dst, N + (ar % (N2 - N))]).reshape(-1, CH)
    w = jnp.concatenate(
        [edge_weight, jnp.zeros((pad,), jnp.float32)]).reshape(-1, CH)

    AGG, HIST = _segsum_sc(Y2, src, dst, w)
    out = _finish_tc(AGG, HIST.reshape(NC * NS, N2), S)
    return out[:N]
